# baseline (speedup 1.0000x reference)
import functools

import jax
import jax.numpy as jnp
from jax import lax
from jax.experimental import pallas as pl
from jax.experimental.pallas import tpu as pltpu
from jax.experimental.pallas import tpu_sc as plsc

_NC = 2
_NS = 16
_NW = _NC * _NS
_L = 16
_UNROLL = 5

_LC5 = (0.0001415121753789439, 0.9954273382579881, -0.4640725804471214,
        0.21641043832781495, -0.05486285286206372)


def _sigmoid_terms(z):
    a = jnp.abs(z)
    t = jnp.exp(-a)
    q = 1.0 / (1.0 + t)
    pf = _LC5[4]
    for c in (_LC5[3], _LC5[2], _LC5[1], _LC5[0]):
        pf = pf * t + c
    ent = pf + (1.0 - q) * a
    p = jnp.where(z < 0.0, 1.0 - q, q)
    return p, ent


def _sigmoid_only(z):
    zc = jnp.maximum(z, -80.0)
    return 1.0 / (1.0 + jnp.exp(-zc))


def _stage1_body(em_hbm, src_hbm, dst_hbm, x_hbm, nfm_hbm, w_hbm, out_hbm,
                 emv, dstv, srcv, hb, nfmv, wv, row1, accr, stage, tmpi,
                 sem, sma, smb, smc, smd, sme,
                 *, ch, d):
    wid = lax.axis_index("c") * _NS + lax.axis_index("s")
    base = wid * ch
    nvec = ch // _L
    ndg = d // _L
    cap = nvec

    c_em = pltpu.async_copy(em_hbm.at[pl.ds(base, ch)], emv, sma)
    c_dst = pltpu.async_copy(dst_hbm.at[pl.ds(base, ch)], dstv, smb)
    c_src = pltpu.async_copy(src_hbm.at[pl.ds(base, ch)], srcv, smc)
    c_nfm = pltpu.async_copy(nfm_hbm, nfmv, smd)
    c_w = pltpu.async_copy(w_hbm, wv, sme)

    for k in range(ndg):
        accr[pl.ds(k * _L, _L)] = jnp.zeros((_L,), jnp.float32)

    zf = jnp.zeros((_L,), jnp.float32)
    zi = jnp.zeros((_L,), jnp.int32)
    iota16 = lax.broadcasted_iota(jnp.int32, (_L,), 0)
    lane_base = iota16 * cap

    nun = nvec // _UNROLL

    c_em.wait()
    c_dst.wait()
    c_src.wait()

    def body(i, carry):
        cnt, s1s, s2s = carry
        s1s = list(s1s)
        s2s = list(s2s)
        for uu in range(_UNROLL):
            off = (i * _UNROLL + uu) * _L
            z = emv[pl.ds(off, _L)]
            p, ent = _sigmoid_terms(z)
            m = dstv[pl.ds(off, _L)] == 0
            slot = lane_base + cnt
            plsc.store_scatter(hb, [slot], off + iota16, mask=m)
            cnt = cnt + jnp.where(m, 1, 0)
            s1s[uu] = s1s[uu] + p
            s2s[uu] = s2s[uu] + ent
        return cnt, tuple(s1s), tuple(s2s)

    cnt, s1s, s2s = lax.fori_loop(
        0, nun, body, (zi, (zf,) * _UNROLL, (zf,) * _UNROLL))
    s1v = s1s[0] + s1s[1] + s1s[2] + s1s[3] + s1s[4]
    s2v = s2s[0] + s2s[1] + s2s[2] + s2s[3] + s2s[4]

    def dbody_for(l):
        def dbody(t, c):
            pos = jnp.full((_L,), l * cap + t, jnp.int32)
            ev = plsc.load_gather(hb, [pos])
            tmpi[...] = plsc.load_gather(srcv, [ev])
            zv = plsc.load_gather(emv, [ev])
            pltpu.async_copy(x_hbm.at[tmpi.at[pl.ds(0, 1)]], row1, sem).wait()
            wvv = _sigmoid_only(zv)
            for k in range(ndg):
                plsc.addupdate(accr.at[pl.ds(k * _L, _L)],
                               wvv * row1[0, pl.ds(k * _L, _L)])
            return c
        return dbody

    for l in range(_L):
        lax.fori_loop(0, cnt[l], dbody_for(l), jnp.int32(0))

    c_nfm.wait()
    c_w.wait()
    logits = zf
    for k in range(ndg):
        fmk = _sigmoid_only(nfmv[pl.ds(k * _L, _L)])
        aggk = accr[pl.ds(k * _L, _L)] * fmk
        for j in range(_L):
            logits = logits + aggk[j] * wv[k * _L + j]

    stage[pl.ds(0, _L)] = s1v
    stage[pl.ds(_L, _L)] = s2v
    stage[pl.ds(2 * _L, _L)] = logits
    pltpu.sync_copy(stage, out_hbm.at[wid])


def _epilogue_body(part_ref, nfm_ref, lab_ref, out_ref, *, e, d):
    eps = 1e-15
    part = part_ref[...]
    s1 = jnp.sum(part[:, 0:_L])
    s2 = jnp.sum(part[:, _L:2 * _L])
    logits = jnp.sum(part[:, 2 * _L:3 * _L], axis=0)

    nfm = nfm_ref[...]
    fm = 1.0 / (1.0 + jnp.exp(-nfm))
    ent2 = -fm * jnp.log(fm + eps) - (1.0 - fm) * jnp.log(1.0 - fm + eps)

    mx = jnp.max(logits)
    sh = logits - mx
    logsm = sh - jnp.log(jnp.sum(jnp.exp(sh)))
    lab = lab_ref[0, 0]
    ii = lax.broadcasted_iota(jnp.int32, (_L,), 0)
    pick = jnp.sum(jnp.where(ii == lab, logsm, 0.0))

    loss = (-pick
            + 0.005 * s1
            + s2 * (1.0 / float(e))
            + jnp.sum(fm)
            + jnp.sum(ent2) * (0.1 / float(d)))
    out_ref[...] = jnp.full((1, 1), loss, jnp.float32)


def kernel(x, edge_index, pred_label, node_feat_mask, edge_mask, W):
    e = edge_mask.shape[0]
    d = x.shape[1]
    c = W.shape[1]
    ch = e // _NW

    mesh = plsc.VectorSubcoreMesh(core_axis_name="c", subcore_axis_name="s",
                                  num_cores=_NC, num_subcores=_NS)
    f32 = jnp.float32
    params = pltpu.CompilerParams(needs_layout_passes=False)

    stage1 = pl.kernel(
        functools.partial(_stage1_body, ch=ch, d=d),
        out_type=jax.ShapeDtypeStruct((_NW, 3 * _L), f32),
        mesh=mesh,
        compiler_params=params,
        scratch_types=[
            pltpu.VMEM((ch,), f32),
            pltpu.VMEM((ch,), jnp.int32),
            pltpu.VMEM((ch,), jnp.int32),
            pltpu.VMEM((ch,), jnp.int32),
            pltpu.VMEM((d,), f32),
            pltpu.VMEM((d, c), f32),
            pltpu.VMEM((1, d), f32),
            pltpu.VMEM((d,), f32),
            pltpu.VMEM((3 * _L,), f32),
            pltpu.VMEM((_L,), jnp.int32),
            pltpu.SemaphoreType.DMA,
            pltpu.SemaphoreType.DMA,
            pltpu.SemaphoreType.DMA,
            pltpu.SemaphoreType.DMA,
            pltpu.SemaphoreType.DMA,
            pltpu.SemaphoreType.DMA,
        ],
    )
    part = stage1(edge_mask, edge_index[0], edge_index[1], x,
                  node_feat_mask, W)

    epilogue = pl.pallas_call(
        functools.partial(_epilogue_body, e=e, d=d),
        out_shape=jax.ShapeDtypeStruct((1, 1), f32),
    )
    out = epilogue(part, node_feat_mask.reshape(1, d),
                   pred_label[:1].reshape(1, 1))
    return out[0, 0]
